# baseline (speedup 1.0000x reference)
import functools

import jax
import jax.numpy as jnp
from jax import lax
from jax.experimental import pallas as pl
from jax.experimental.pallas import tpu as pltpu
from jax.experimental.pallas import tpu_sc as plsc

_NC = 2
_NS = 16
_LANES = 16


def _fold_w1_tensorcore(emb_table, W1):
    V, D = emb_table.shape
    H = W1.shape[1]
    BLK = 8192
    grid = (V + BLK - 1) // BLK

    def body(tT_ref, w1_ref, o_ref):
        o_ref[:] = jax.lax.dot_general(
            tT_ref[:], w1_ref[:], (((0,), (0,)), ((), ())),
            preferred_element_type=jnp.float32)

    return pl.pallas_call(
        body,
        grid=(grid,),
        in_specs=[
            pl.BlockSpec((D, BLK), lambda i: (0, i)),
            pl.BlockSpec((D, H), lambda i: (0, 0)),
        ],
        out_specs=pl.BlockSpec((BLK, H), lambda i: (i, 0)),
        out_shape=jax.ShapeDtypeStruct((V, H), jnp.float32),
    )(emb_table.T, W1)


def _cbow_sparsecore(indices, tablew):
    B, L = indices.shape
    V, H = tablew.shape
    NW = _NC * _NS
    BPW = B // NW
    NCD = H // _LANES
    CB = 2
    ROWS = CB * L
    NCH = BPW // CB

    mesh = plsc.VectorSubcoreMesh(
        core_axis_name="c", subcore_axis_name="s",
        num_cores=_NC, num_subcores=_NS)

    NBUF = 2

    @functools.partial(
        pl.kernel,
        mesh=mesh,
        out_type=jax.ShapeDtypeStruct((B, H), jnp.float32),
        scratch_types=[
            pltpu.VMEM((NCH, ROWS), jnp.int32),
            pltpu.VMEM((NBUF, ROWS, H), jnp.float32),
            pltpu.VMEM((BPW // 2, H), jnp.float32),
            [pltpu.SemaphoreType.DMA] * NBUF,
        ],
        compiler_params=pltpu.CompilerParams(use_tc_tiling_on_sc=False),
    )
    def cbow_kernel(idx_hbm, table_hbm, out_hbm, idx_v, rows_v, acc_v, sems):
        wid = lax.axis_index("s") * _NC + lax.axis_index("c")
        HC = NCH // 2
        pltpu.sync_copy(idx_hbm.at[pl.ds(wid * NCH, NCH)], idx_v)

        def start(j, b):
            pltpu.async_copy(table_hbm.at[idx_v.at[j]], rows_v.at[b], sems[b])

        def finish(j, jc, b):
            pltpu.make_async_copy(
                table_hbm.at[idx_v.at[j]], rows_v.at[b], sems[b]).wait()
            for be in range(CB):
                for cg in range(0, NCD, 4):
                    accs = [rows_v[b, be * L, pl.ds((cg + c) * _LANES, _LANES)]
                            for c in range(4)]
                    for r in range(1, L):
                        for c in range(4):
                            accs[c] = accs[c] + rows_v[
                                b, be * L + r, pl.ds((cg + c) * _LANES, _LANES)]
                    for c in range(4):
                        acc_v[jc * CB + be,
                              pl.ds((cg + c) * _LANES, _LANES)] = accs[c]

        @pl.loop(0, 2)
        def half(hh):
            base = hh * HC
            for b in range(NBUF):
                start(base + b, b)

            @pl.loop(0, HC - NBUF, step=NBUF)
            def chunk(j0):
                for b in range(NBUF):
                    finish(base + j0 + b, j0 + b, b)
                    start(base + j0 + b + NBUF, b)

            for b in range(NBUF):
                finish(base + HC - NBUF + b, HC - NBUF + b, b)

            pltpu.sync_copy(
                acc_v, out_hbm.at[pl.ds((wid * NCH + base) * CB, HC * CB)])

    idx2 = indices.reshape(B // CB, ROWS)
    return cbow_kernel(idx2, tablew)


def _ffn_tensorcore(x, b1, W2, b2):
    B, H = x.shape
    C = W2.shape[1]
    BB = 2048

    def body(x_ref, b1_ref, w2_ref, b2_ref, o_ref):
        h = jnp.maximum(x_ref[:] + b1_ref[:], 0.0)
        o_ref[:] = jnp.dot(h, w2_ref[:],
                           preferred_element_type=jnp.float32) + b2_ref[:]

    return pl.pallas_call(
        body,
        grid=(B // BB,),
        in_specs=[
            pl.BlockSpec((BB, H), lambda i: (i, 0)),
            pl.BlockSpec((1, H), lambda i: (0, 0)),
            pl.BlockSpec((H, C), lambda i: (0, 0)),
            pl.BlockSpec((1, C), lambda i: (0, 0)),
        ],
        out_specs=pl.BlockSpec((BB, C), lambda i: (i, 0)),
        out_shape=jax.ShapeDtypeStruct((B, C), jnp.float32),
    )(x, b1.reshape(1, H), W2, b2.reshape(1, C))


def kernel(indices, emb_table, W1, b1, W2, b2):
    tablew = _fold_w1_tensorcore(emb_table, W1)
    hpre = _cbow_sparsecore(indices, tablew)
    return _ffn_tensorcore(hpre, b1, W2, b2)


# speedup vs baseline: 1.1641x; 1.1641x over previous
import functools

import jax
import jax.numpy as jnp
from jax import lax
from jax.experimental import pallas as pl
from jax.experimental.pallas import tpu as pltpu
from jax.experimental.pallas import tpu_sc as plsc

_NC = 2
_NS = 16
_LANES = 16


def _fold_w1_tensorcore(emb_table, W1):
    V, D = emb_table.shape
    H = W1.shape[1]
    BLK = 16384
    grid = (V + BLK - 1) // BLK

    def body(tT_ref, w1_ref, o_ref):
        o_ref[:] = jax.lax.dot_general(
            tT_ref[:], w1_ref[:], (((0,), (0,)), ((), ())),
            preferred_element_type=jnp.float32)

    return pl.pallas_call(
        body,
        grid=(grid,),
        in_specs=[
            pl.BlockSpec((D, BLK), lambda i: (0, i)),
            pl.BlockSpec((D, H), lambda i: (0, 0)),
        ],
        out_specs=pl.BlockSpec((BLK, H), lambda i: (i, 0)),
        out_shape=jax.ShapeDtypeStruct((V, H), jnp.float32),
    )(emb_table.T, W1)


def _cbow_sparsecore(indices, tablew):
    B, L = indices.shape
    V, H = tablew.shape
    NW = _NC * _NS
    BPW = B // NW
    NCD = H // _LANES

    mesh = plsc.VectorSubcoreMesh(
        core_axis_name="c", subcore_axis_name="s",
        num_cores=_NC, num_subcores=_NS)

    NBUF = 2

    @functools.partial(
        pl.kernel,
        mesh=mesh,
        out_type=jax.ShapeDtypeStruct((B, H), jnp.float32),
        scratch_types=[
            pltpu.VMEM((BPW, L), jnp.int32),
            pltpu.VMEM((NBUF, L, H), jnp.float32),
            pltpu.VMEM((BPW // 2, H), jnp.float32),
            [pltpu.SemaphoreType.DMA] * NBUF,
        ],
        compiler_params=pltpu.CompilerParams(use_tc_tiling_on_sc=False),
    )
    def cbow_kernel(idx_hbm, table_hbm, out_hbm, idx_v, rows_v, acc_v, sems):
        wid = lax.axis_index("s") * _NC + lax.axis_index("c")
        HB = BPW // 2
        pltpu.sync_copy(idx_hbm.at[pl.ds(wid * BPW, BPW)], idx_v)

        def start(j, b):
            pltpu.async_copy(table_hbm.at[idx_v.at[j]], rows_v.at[b], sems[b])

        def finish(j, jout, b):
            pltpu.make_async_copy(
                table_hbm.at[idx_v.at[j]], rows_v.at[b], sems[b]).wait()
            for cg in range(0, NCD, 4):
                accs = [rows_v[b, 0, pl.ds((cg + c) * _LANES, _LANES)]
                        for c in range(4)]
                for r in range(1, L):
                    for c in range(4):
                        accs[c] = accs[c] + rows_v[
                            b, r, pl.ds((cg + c) * _LANES, _LANES)]
                for c in range(4):
                    acc_v[jout, pl.ds((cg + c) * _LANES, _LANES)] = accs[c]

        @pl.loop(0, 2)
        def half(hh):
            base = hh * HB
            for b in range(NBUF):
                start(base + b, b)

            @pl.loop(0, HB - NBUF, step=NBUF)
            def chunk(j0):
                for b in range(NBUF):
                    finish(base + j0 + b, j0 + b, b)
                    start(base + j0 + b + NBUF, b)

            for b in range(NBUF):
                finish(base + HB - NBUF + b, HB - NBUF + b, b)

            pltpu.sync_copy(acc_v, out_hbm.at[pl.ds(wid * BPW + base, HB)])

    return cbow_kernel(indices, tablew)


def _ffn_tensorcore(x, b1, W2, b2):
    B, H = x.shape
    C = W2.shape[1]
    BB = 2048

    def body(x_ref, b1_ref, w2_ref, b2_ref, o_ref):
        h = jnp.maximum(x_ref[:] + b1_ref[:], 0.0)
        o_ref[:] = jnp.dot(h, w2_ref[:],
                           preferred_element_type=jnp.float32) + b2_ref[:]

    return pl.pallas_call(
        body,
        grid=(B // BB,),
        in_specs=[
            pl.BlockSpec((BB, H), lambda i: (i, 0)),
            pl.BlockSpec((1, H), lambda i: (0, 0)),
            pl.BlockSpec((H, C), lambda i: (0, 0)),
            pl.BlockSpec((1, C), lambda i: (0, 0)),
        ],
        out_specs=pl.BlockSpec((BB, C), lambda i: (i, 0)),
        out_shape=jax.ShapeDtypeStruct((B, C), jnp.float32),
    )(x, b1.reshape(1, H), W2, b2.reshape(1, C))


def kernel(indices, emb_table, W1, b1, W2, b2):
    tablew = _fold_w1_tensorcore(emb_table, W1)
    hpre = _cbow_sparsecore(indices, tablew)
    return _ffn_tensorcore(hpre, b1, W2, b2)


# speedup vs baseline: 1.1759x; 1.0101x over previous
import functools

import jax
import jax.numpy as jnp
from jax import lax
from jax.experimental import pallas as pl
from jax.experimental.pallas import tpu as pltpu
from jax.experimental.pallas import tpu_sc as plsc

_NC = 2
_NS = 16
_LANES = 16


def _fold_w1_tensorcore(emb_table, W1):
    V, D = emb_table.shape
    H = W1.shape[1]
    BLK = 32768
    grid = (V + BLK - 1) // BLK

    def body(tT_ref, w1_ref, o_ref):
        o_ref[:] = jax.lax.dot_general(
            tT_ref[:], w1_ref[:], (((0,), (0,)), ((), ())),
            preferred_element_type=jnp.float32)

    return pl.pallas_call(
        body,
        grid=(grid,),
        in_specs=[
            pl.BlockSpec((D, BLK), lambda i: (0, i)),
            pl.BlockSpec((D, H), lambda i: (0, 0)),
        ],
        out_specs=pl.BlockSpec((BLK, H), lambda i: (i, 0)),
        out_shape=jax.ShapeDtypeStruct((V, H), jnp.float32),
    )(emb_table.T, W1)


def _cbow_sparsecore(indices, tablew):
    B, L = indices.shape
    V, H = tablew.shape
    NW = _NC * _NS
    BPW = B // NW
    NCD = H // _LANES

    mesh = plsc.VectorSubcoreMesh(
        core_axis_name="c", subcore_axis_name="s",
        num_cores=_NC, num_subcores=_NS)

    NBUF = 2

    @functools.partial(
        pl.kernel,
        mesh=mesh,
        out_type=jax.ShapeDtypeStruct((B, H), jnp.float32),
        scratch_types=[
            pltpu.VMEM((BPW, L), jnp.int32),
            pltpu.VMEM((NBUF, L, H), jnp.float32),
            pltpu.VMEM((BPW // 2, H), jnp.float32),
            [pltpu.SemaphoreType.DMA] * NBUF,
        ],
        compiler_params=pltpu.CompilerParams(use_tc_tiling_on_sc=False),
    )
    def cbow_kernel(idx_hbm, table_hbm, out_hbm, idx_v, rows_v, acc_v, sems):
        wid = lax.axis_index("s") * _NC + lax.axis_index("c")
        HB = BPW // 2
        pltpu.sync_copy(idx_hbm.at[pl.ds(wid * BPW, BPW)], idx_v)

        def start(j, b):
            pltpu.async_copy(table_hbm.at[idx_v.at[j]], rows_v.at[b], sems[b])

        def finish(j, jout, b):
            pltpu.make_async_copy(
                table_hbm.at[idx_v.at[j]], rows_v.at[b], sems[b]).wait()
            for cg in range(0, NCD, 4):
                accs = [rows_v[b, 0, pl.ds((cg + c) * _LANES, _LANES)]
                        for c in range(4)]
                for r in range(1, L):
                    for c in range(4):
                        accs[c] = accs[c] + rows_v[
                            b, r, pl.ds((cg + c) * _LANES, _LANES)]
                for c in range(4):
                    acc_v[jout, pl.ds((cg + c) * _LANES, _LANES)] = accs[c]

        @pl.loop(0, 2)
        def half(hh):
            base = hh * HB
            for b in range(NBUF):
                start(base + b, b)

            @pl.loop(0, HB - NBUF, step=NBUF)
            def chunk(j0):
                for b in range(NBUF):
                    finish(base + j0 + b, j0 + b, b)
                    start(base + j0 + b + NBUF, b)

            for b in range(NBUF):
                finish(base + HB - NBUF + b, HB - NBUF + b, b)

            pltpu.sync_copy(acc_v, out_hbm.at[pl.ds(wid * BPW + base, HB)])

    return cbow_kernel(indices, tablew)


def _ffn_tensorcore(x, b1, W2, b2):
    B, H = x.shape
    C = W2.shape[1]
    BB = 2048

    def body(x_ref, b1_ref, w2_ref, b2_ref, o_ref):
        h = jnp.maximum(x_ref[:] + b1_ref[:], 0.0)
        o_ref[:] = jnp.dot(h, w2_ref[:],
                           preferred_element_type=jnp.float32) + b2_ref[:]

    return pl.pallas_call(
        body,
        grid=(B // BB,),
        in_specs=[
            pl.BlockSpec((BB, H), lambda i: (i, 0)),
            pl.BlockSpec((1, H), lambda i: (0, 0)),
            pl.BlockSpec((H, C), lambda i: (0, 0)),
            pl.BlockSpec((1, C), lambda i: (0, 0)),
        ],
        out_specs=pl.BlockSpec((BB, C), lambda i: (i, 0)),
        out_shape=jax.ShapeDtypeStruct((B, C), jnp.float32),
    )(x, b1.reshape(1, H), W2, b2.reshape(1, C))


def kernel(indices, emb_table, W1, b1, W2, b2):
    tablew = _fold_w1_tensorcore(emb_table, W1)
    hpre = _cbow_sparsecore(indices, tablew)
    return _ffn_tensorcore(hpre, b1, W2, b2)
